# baseline (speedup 1.0000x reference)
import functools
import jax
import jax.numpy as jnp
from jax import lax
from jax.experimental import pallas as pl
from jax.experimental.pallas import tpu as pltpu
from jax.experimental.pallas import tpu_sc as plsc

SC_N = 32768
SC_K = 16384
SC_ROWS = 1024
SC_NV = SC_N // 16
SC_NW = 32
SC_RPW = SC_ROWS // SC_NW
SC_P = 4
SC_NVP = SC_NV // SC_P
SC_NB = 256 * 16 * SC_P
SC_NBV = SC_NB // 16


def _key(bits):
    m = lax.shift_right_arithmetic(bits, 31)
    return bits ^ ((m ^ -1) & 0x7FFFFFFF)


def _logical_to_flat(pos):
    vv = pos & (SC_NVP - 1)
    pp = lax.shift_right_logical(pos, 9) & (SC_P - 1)
    ll = lax.shift_right_logical(pos, 11)
    return (vv << 6) | (pp << 4) | ll


def _sc_body(x_hbm, out_hbm, bufa, bufb, bufc, hist, sums):
    lane = lax.iota(jnp.int32, 16)
    ones = jnp.ones((16,), jnp.int32)
    mask0 = lane == 0
    wid = lax.axis_index("s") * 2 + lax.axis_index("c")

    def radix_pass(src, dst, shift, transform):
        def cl(i, carry):
            hist[pl.ds(i * 16, 16)] = jnp.zeros((16,), jnp.int32)
            return carry
        lax.fori_loop(0, SC_NBV, cl, 0, unroll=8)

        def ha(v, carry):
            k = src[pl.ds(v * 16, 16)]
            if transform:
                k = _key(k)
            dig = lax.shift_right_logical(k, shift) & 0xFF
            idx = dig * 64 + lane * SC_P + (v & (SC_P - 1))
            cnt = plsc.load_gather(hist, [idx])
            plsc.store_scatter(hist, [idx], cnt + ones)
            return carry
        lax.fori_loop(0, SC_NV, ha, 0, unroll=8)

        def pa(i, carry):
            h = hist[pl.ds(i * 16, 16)]
            tot = jnp.sum(h)
            plsc.store_scatter(
                sums, [jnp.zeros((16,), jnp.int32) + i],
                jnp.zeros((16,), jnp.int32) + tot, mask=mask0)
            return carry
        lax.fori_loop(0, SC_NBV, pa, 0, unroll=4)

        def pb(i, s_carry):
            s = sums[pl.ds(i * 16, 16)]
            c = plsc.cumsum(s)
            tot = jnp.max(c)
            sums[pl.ds(i * 16, 16)] = (c - s) + s_carry
            return s_carry + tot
        lax.fori_loop(0, SC_NBV // 16, pb, jnp.int32(0))

        def pc(i, carry):
            sv = sums[pl.ds(i * 16, 16)]
            for j2 in range(16):
                oh = (lane == j2).astype(jnp.int32)
                bs = jnp.sum(sv * oh)
                hv = i * 16 + j2
                h = hist[pl.ds(hv * 16, 16)]
                c = plsc.cumsum(h)
                hist[pl.ds(hv * 16, 16)] = (c - h) + bs
            return carry
        lax.fori_loop(0, SC_NBV // 16, pc, 0)

        def pm(v, carry):
            k = src[pl.ds(v * 16, 16)]
            if transform:
                k = _key(k)
            dig = lax.shift_right_logical(k, shift) & 0xFF
            idx = dig * 64 + lane * SC_P + (v & (SC_P - 1))
            pos = plsc.load_gather(hist, [idx])
            plsc.store_scatter(hist, [idx], pos + ones)
            plsc.store_scatter(dst, [_logical_to_flat(pos)], k)
            return carry
        lax.fori_loop(0, SC_NV, pm, 0, unroll=8)

    def row_loop(ri, carry):
        r = wid * SC_RPW + ri
        pltpu.sync_copy(x_hbm.at[r], bufa)
        radix_pass(bufa, bufb, 0, True)
        radix_pass(bufb, bufc, 8, False)
        radix_pass(bufc, bufb, 16, False)
        radix_pass(bufb, bufc, 24, False)

        def go(j, c2):
            jj = j * 16 + lane
            k = plsc.load_gather(bufc, [_logical_to_flat(jj)])
            bufa[pl.ds(j * 16, 16)] = _key(k)
            return c2
        lax.fori_loop(0, SC_K // 16, go, 0, unroll=8)
        pltpu.sync_copy(bufa.at[pl.ds(0, SC_K)], out_hbm.at[r])
        return carry

    lax.fori_loop(0, SC_RPW, row_loop, 0)


@functools.partial(
    pl.kernel,
    out_type=jax.ShapeDtypeStruct((SC_ROWS, SC_K), jnp.int32),
    mesh=plsc.VectorSubcoreMesh(core_axis_name="c", subcore_axis_name="s"),
    scratch_types=[
        pltpu.VMEM((SC_N,), jnp.int32),
        pltpu.VMEM((SC_N,), jnp.int32),
        pltpu.VMEM((SC_N,), jnp.int32),
        pltpu.VMEM((SC_NB,), jnp.int32),
        pltpu.VMEM((SC_NBV,), jnp.int32),
    ],
    compiler_params=pltpu.CompilerParams(needs_layout_passes=False),
)
def _sc_topk(x_hbm, out_hbm, bufa, bufb, bufc, hist, sums):
    _sc_body(x_hbm, out_hbm, bufa, bufb, bufc, hist, sums)


@jax.jit
def kernel(x, layer_idx):
    b, ch, n = x.shape
    xb = lax.bitcast_convert_type(x.reshape(b * ch, n), jnp.int32)
    outb = _sc_topk(xb)
    out = lax.bitcast_convert_type(outb, jnp.float32).reshape(b, ch, n // 2)
    return out + jnp.zeros((), dtype=out.dtype) * layer_idx


# speedup vs baseline: 3.0945x; 3.0945x over previous
import functools
import jax
import jax.numpy as jnp
from jax import lax
from jax.experimental import pallas as pl
from jax.experimental.pallas import tpu as pltpu
from jax.experimental.pallas import tpu_sc as plsc

SC_N = 32768
SC_K = 16384
SC_ROWS = 384
SC_NV = SC_N // 16
SC_NW = 32
SC_RPW = SC_ROWS // SC_NW


def _key(bits):
    m = lax.shift_right_arithmetic(bits, 31)
    return bits ^ ((m ^ -1) & 0x7FFFFFFF)


def _sc_body(x_hbm, out_hbm, bufa, bufb, bufc, hist):
    lane = lax.iota(jnp.int32, 16)
    ones = jnp.ones((16,), jnp.int32)
    wid = lax.axis_index("s") * 2 + lax.axis_index("c")

    def radix_pass(src, dst, shift, transform):
        def cl(i, carry):
            hist[pl.ds(i * 16, 16)] = jnp.zeros((16,), jnp.int32)
            return carry
        lax.fori_loop(0, 256, cl, 0, unroll=8)

        def ha(v, carry):
            k = src[pl.ds(v * 16, 16)]
            if transform:
                k = _key(k)
            dig = lax.shift_right_logical(k, shift) & 0xFF
            idx = dig * 16 + lane
            cnt = plsc.load_gather(hist, [idx])
            plsc.store_scatter(hist, [idx], cnt + ones)
            return carry
        lax.fori_loop(0, SC_NV, ha, 0, unroll=8)

        def pf(b, s_carry):
            row = hist[pl.ds(b * 16, 16)]
            csum = plsc.cumsum(row)
            tot = jnp.max(csum)
            hist[pl.ds(b * 16, 16)] = (csum - row) + s_carry
            return s_carry + tot
        lax.fori_loop(0, 256, pf, jnp.int32(0), unroll=4)

        def pm(v, carry):
            k = src[pl.ds(v * 16, 16)]
            if transform:
                k = _key(k)
            dig = lax.shift_right_logical(k, shift) & 0xFF
            idx = dig * 16 + lane
            pos = plsc.load_gather(hist, [idx])
            plsc.store_scatter(hist, [idx], pos + 1)
            didx = ((pos & (SC_NV - 1)) << 4) | lax.shift_right_logical(pos, 11)
            plsc.store_scatter(dst, [didx], k)
            return carry
        lax.fori_loop(0, SC_NV, pm, 0, unroll=8)

    def row_loop(ri, carry):
        r = wid * SC_RPW + ri
        pltpu.sync_copy(x_hbm.at[r], bufa)
        radix_pass(bufa, bufb, 0, True)
        radix_pass(bufb, bufc, 8, False)
        radix_pass(bufc, bufb, 16, False)
        radix_pass(bufb, bufc, 24, False)

        def go(j, c2):
            jj = j * 16 + lane
            src_idx = ((jj & (SC_NV - 1)) << 4) | lax.shift_right_logical(jj, 11)
            k = plsc.load_gather(bufc, [src_idx])
            bufa[pl.ds(j * 16, 16)] = _key(k)
            return c2
        lax.fori_loop(0, SC_K // 16, go, 0, unroll=8)
        pltpu.sync_copy(bufa.at[pl.ds(0, SC_K)], out_hbm.at[r])
        return carry

    lax.fori_loop(0, SC_RPW, row_loop, 0)


@functools.partial(
    pl.kernel,
    out_type=jax.ShapeDtypeStruct((SC_ROWS, SC_K), jnp.int32),
    mesh=plsc.VectorSubcoreMesh(core_axis_name="c", subcore_axis_name="s"),
    scratch_types=[
        pltpu.VMEM((SC_N,), jnp.int32),
        pltpu.VMEM((SC_N,), jnp.int32),
        pltpu.VMEM((SC_N,), jnp.int32),
        pltpu.VMEM((4096,), jnp.int32),
    ],
    compiler_params=pltpu.CompilerParams(needs_layout_passes=False),
)
def _sc_topk(x_hbm, out_hbm, bufa, bufb, bufc, hist):
    _sc_body(x_hbm, out_hbm, bufa, bufb, bufc, hist)


N = 32768
TOPK = 16384
LOGN = 15
CB = 128
CH = 256


def _substage_big(buf, p, j, nr):
    s = 1 << j
    n_iter = (nr // 2) // CH

    def body(t, carry):
        tb = t * CH
        off = tb & (s - 1)
        rowa = ((tb >> j) << (j + 1)) | off
        a = buf[pl.ds(rowa, CH), :]
        b = buf[pl.ds(rowa + s, CH), :]
        hi = jnp.maximum(a, b)
        lo = jnp.minimum(a, b)
        if p is None:
            rhi = rowa
            rlo = rowa + s
        else:
            bs = ((rowa >> p) & 1) << j
            rhi = rowa + bs
            rlo = rowa + s - bs
        buf[pl.ds(rhi, CH), :] = hi
        buf[pl.ds(rlo, CH), :] = lo
        return carry

    jax.lax.fori_loop(0, n_iter, body, 0)


def _small_stages(buf, p, jhi, nr):
    n_iter = nr // CH

    def body(t, carry):
        r0 = t * CH
        c = buf[pl.ds(r0, CH), :]
        if p is None:
            sgn = None
        elif (1 << (p + 1)) <= CH:
            rows = jax.lax.broadcasted_iota(jnp.int32, (CH, 1), 0)
            sgn = jnp.where(((rows >> p) & 1) == 0, 1.0, -1.0).astype(
                jnp.float32)
        else:
            bit = (r0 >> p) & 1
            sgn = (1 - 2 * bit).astype(jnp.float32)
        if sgn is not None:
            c = c * sgn
        pieces = [c]
        for j in range(jhi, 2, -1):
            s = 1 << j
            new_pieces = []
            for piece in pieces:
                for base in range(0, piece.shape[0], 2 * s):
                    a = piece[base:base + s]
                    b = piece[base + s:base + 2 * s]
                    new_pieces.append(jnp.maximum(a, b))
                    new_pieces.append(jnp.minimum(a, b))
            pieces = new_pieces
        c = jnp.concatenate(pieces, axis=0) if len(pieces) > 1 else pieces[0]
        for j in range(min(jhi, 2), -1, -1):
            s = 1 << j
            cb = c.reshape(CH // (2 * s), 2, s, CB)
            hi = jnp.maximum(cb[:, 0], cb[:, 1])
            lo = jnp.minimum(cb[:, 0], cb[:, 1])
            c = jnp.concatenate(
                [hi[:, None], lo[:, None]], axis=1).reshape(CH, CB)
        if sgn is not None:
            c = c * sgn
        buf[pl.ds(r0, CH), :] = c
        return carry

    jax.lax.fori_loop(0, n_iter, body, 0)


def _half_merge(buf):
    def body(t, carry):
        r0 = t * CH
        a = buf[pl.ds(r0, CH), :]
        b = buf[pl.ds(r0 + TOPK, CH), :]
        buf[pl.ds(r0, CH), :] = jnp.maximum(a, b)
        return carry

    jax.lax.fori_loop(0, TOPK // CH, body, 0)


def _sort_body(x_hbm, o_hbm, buf, sem_in, sem_out):
    i = pl.program_id(0)
    cin = pltpu.make_async_copy(
        x_hbm.at[:, pl.ds(i * CB, CB)], buf, sem_in)
    cin.start()
    cin.wait()

    lc = CH.bit_length() - 1
    for p in range(1, LOGN):
        for j in range(p - 1, lc - 1, -1):
            _substage_big(buf, p, j, N)
        _small_stages(buf, p, min(p - 1, lc - 1), N)
    _half_merge(buf)
    for j in range(LOGN - 2, lc - 1, -1):
        _substage_big(buf, None, j, TOPK)
    _small_stages(buf, None, lc - 1, TOPK)

    cout = pltpu.make_async_copy(
        buf.at[pl.ds(0, TOPK), :], o_hbm.at[:, pl.ds(i * CB, CB)], sem_out)
    cout.start()
    cout.wait()


def _topk_columns(xt):
    n, r = xt.shape
    return pl.pallas_call(
        _sort_body,
        grid=(r // CB,),
        in_specs=[pl.BlockSpec(memory_space=pl.ANY)],
        out_specs=pl.BlockSpec(memory_space=pl.ANY),
        out_shape=jax.ShapeDtypeStruct((n // 2, r), jnp.float32),
        scratch_shapes=[
            pltpu.VMEM((N, CB), jnp.float32),
            pltpu.SemaphoreType.DMA,
            pltpu.SemaphoreType.DMA,
        ],
        compiler_params=pltpu.CompilerParams(
            dimension_semantics=("arbitrary",),
        ),
    )(xt)


@jax.jit
def kernel(x, layer_idx):
    b, ch, n = x.shape
    x2d = x.reshape(b * ch, n)
    xb = lax.bitcast_convert_type(x2d[:SC_ROWS], jnp.int32)
    out_sc = lax.bitcast_convert_type(_sc_topk(xb), jnp.float32)
    xt = x2d[SC_ROWS:].T
    out_tc = _topk_columns(xt).T
    out = jnp.concatenate([out_sc, out_tc], axis=0).reshape(b, ch, n // 2)
    return out + jnp.zeros((), dtype=out.dtype) * layer_idx


# speedup vs baseline: 4.4119x; 1.4257x over previous
import functools
import jax
import jax.numpy as jnp
from jax import lax
from jax.experimental import pallas as pl
from jax.experimental.pallas import tpu as pltpu
from jax.experimental.pallas import tpu_sc as plsc

SC_N = 32768
SC_K = 16384
SC_ROWS = 256
SC_NV = SC_N // 16
SC_NW = 32
SC_RPW = SC_ROWS // SC_NW


def _key(bits):
    m = lax.shift_right_arithmetic(bits, 31)
    return bits ^ ((m ^ -1) & 0x7FFFFFFF)


def _sc_body(x_hbm, out_hbm, bufa, bufb, bufc, hist):
    lane = lax.iota(jnp.int32, 16)
    ones = jnp.ones((16,), jnp.int32)
    wid = lax.axis_index("s") * 2 + lax.axis_index("c")

    def radix_pass(src, dst, shift, transform):
        def cl(i, carry):
            hist[pl.ds(i * 16, 16)] = jnp.zeros((16,), jnp.int32)
            return carry
        lax.fori_loop(0, 256, cl, 0, unroll=8)

        def ha(v, carry):
            k = src[pl.ds(v * 16, 16)]
            if transform:
                k = _key(k)
            dig = lax.shift_right_logical(k, shift) & 0xFF
            idx = dig * 16 + lane
            cnt = plsc.load_gather(hist, [idx])
            plsc.store_scatter(hist, [idx], cnt + ones)
            return carry
        lax.fori_loop(0, SC_NV, ha, 0, unroll=8)

        def pf(b, s_carry):
            row = hist[pl.ds(b * 16, 16)]
            csum = plsc.cumsum(row)
            tot = jnp.max(csum)
            hist[pl.ds(b * 16, 16)] = (csum - row) + s_carry
            return s_carry + tot
        lax.fori_loop(0, 256, pf, jnp.int32(0), unroll=4)

        def pm(v, carry):
            k = src[pl.ds(v * 16, 16)]
            if transform:
                k = _key(k)
            dig = lax.shift_right_logical(k, shift) & 0xFF
            idx = dig * 16 + lane
            pos = plsc.load_gather(hist, [idx])
            plsc.store_scatter(hist, [idx], pos + 1)
            didx = ((pos & (SC_NV - 1)) << 4) | lax.shift_right_logical(pos, 11)
            plsc.store_scatter(dst, [didx], k)
            return carry
        lax.fori_loop(0, SC_NV, pm, 0, unroll=8)

    def row_loop(ri, carry):
        r = wid * SC_RPW + ri
        pltpu.sync_copy(x_hbm.at[r], bufa)
        radix_pass(bufa, bufb, 0, True)
        radix_pass(bufb, bufc, 8, False)
        radix_pass(bufc, bufb, 16, False)
        radix_pass(bufb, bufc, 24, False)

        def go(j, c2):
            jj = j * 16 + lane
            src_idx = ((jj & (SC_NV - 1)) << 4) | lax.shift_right_logical(jj, 11)
            k = plsc.load_gather(bufc, [src_idx])
            bufa[pl.ds(j * 16, 16)] = _key(k)
            return c2
        lax.fori_loop(0, SC_K // 16, go, 0, unroll=8)
        pltpu.sync_copy(bufa.at[pl.ds(0, SC_K)], out_hbm.at[r])
        return carry

    lax.fori_loop(0, SC_RPW, row_loop, 0)


@functools.partial(
    pl.kernel,
    out_type=jax.ShapeDtypeStruct((SC_ROWS, SC_K), jnp.int32),
    mesh=plsc.VectorSubcoreMesh(core_axis_name="c", subcore_axis_name="s"),
    scratch_types=[
        pltpu.VMEM((SC_N,), jnp.int32),
        pltpu.VMEM((SC_N,), jnp.int32),
        pltpu.VMEM((SC_N,), jnp.int32),
        pltpu.VMEM((4096,), jnp.int32),
    ],
    compiler_params=pltpu.CompilerParams(needs_layout_passes=False),
)
def _sc_topk(x_hbm, out_hbm, bufa, bufb, bufc, hist):
    _sc_body(x_hbm, out_hbm, bufa, bufb, bufc, hist)


N = 32768
TOPK = 16384
CB = 128
CH = 256


def _xchg(c, steps):
    for s in steps:
        cb = c.reshape(CH // (2 * s), 2, s, CB)
        hi = jnp.maximum(cb[:, 0], cb[:, 1])
        lo = jnp.minimum(cb[:, 0], cb[:, 1])
        c = jnp.concatenate(
            [hi[:, None], lo[:, None]], axis=1).reshape(CH, CB)
    return c


def _sgn_static(bits):
    rows = jax.lax.broadcasted_iota(jnp.int32, (CH, 1), 0)
    acc = rows >> bits[0]
    for b in bits[1:]:
        acc = acc ^ (rows >> b)
    return jnp.where((acc & 1) == 0, 1.0, -1.0).astype(jnp.float32)


def _chunk_pass(buf, nr, steps, pre=None, post=None,
                scalar_bit=None, base=0):
    def body(t, carry):
        r0 = base + t * CH
        c = buf[pl.ds(r0, CH), :]
        if scalar_bit is not None:
            sc = (1 - 2 * ((r0 >> scalar_bit) & 1)).astype(jnp.float32)
            c = c * sc
        if pre is not None:
            c = c * _sgn_static(pre)
        c = _xchg(c, steps)
        if post is not None:
            c = c * _sgn_static(post)
        if scalar_bit is not None:
            c = c * sc
        buf[pl.ds(r0, CH), :] = c
        return carry

    jax.lax.fori_loop(0, nr // CH, body, 0)


def _big_pass(buf, s, nr, pb=None, base=0):
    j = s.bit_length() - 1
    n_iter = (nr // 2) // CH

    def body(t, carry):
        tb = t * CH
        off = tb & (s - 1)
        rowa = base + (((tb >> j) << (j + 1)) | off)
        a = buf[pl.ds(rowa, CH), :]
        b = buf[pl.ds(rowa + s, CH), :]
        hi = jnp.maximum(a, b)
        lo = jnp.minimum(a, b)
        if pb is None:
            rhi = rowa
            rlo = rowa + s
        else:
            bs = ((rowa >> pb) & 1) << j
            rhi = rowa + bs
            rlo = rowa + s - bs
        buf[pl.ds(rhi, CH), :] = hi
        buf[pl.ds(rlo, CH), :] = lo
        return carry

    jax.lax.fori_loop(0, n_iter, body, 0)


def _compact(buf):
    def body(t, carry):
        ci = buf[pl.ds(2 * t * CH, 2 * CH), :]
        cb = ci.reshape(2 * CH // 8, 2, 4, CB)
        m = jnp.maximum(cb[:, 0], cb[:, 1])
        buf[pl.ds(t * CH, CH), :] = m.reshape(CH, CB)
        return carry

    jax.lax.fori_loop(0, TOPK // CH, body, 0)


def _unpermute(buf):
    def body(t, carry):
        g = t >> 4
        w0 = (t << 10) - (g << 14) + g
        win = buf[pl.ds(w0, 4 * CH), :]
        c = win.reshape(CH, 4, CB)[:, 0, :]
        buf[pl.ds(TOPK + t * CH, CH), :] = c
        return carry

    jax.lax.fori_loop(0, TOPK // CH, body, 0)


def _sort_body(x_hbm, o_hbm, buf, sem_in, sem_out):
    i = pl.program_id(0)
    cin = pltpu.make_async_copy(
        x_hbm.at[:, pl.ds(i * CB, CB)], buf, sem_in)
    cin.start()
    cin.wait()

    def s14(t, carry):
        r0 = t * CH
        c = buf[pl.ds(r0, CH), :]
        c = c * _sgn_static([4])
        c = _xchg(c, [8])
        c = c * _sgn_static([4, 5])
        c = _xchg(c, [16, 8])
        c = c * _sgn_static([5, 6])
        c = _xchg(c, [32, 16, 8])
        c = c * _sgn_static([6, 7])
        c = _xchg(c, [64, 32, 16, 8])
        c = c * _sgn_static([7])
        buf[pl.ds(r0, CH), :] = c
        return carry
    jax.lax.fori_loop(0, N // CH, s14, 0)

    for p in range(5, 12):
        pb = p + 3
        for j in range(p - 1, 4, -1):
            _big_pass(buf, 1 << (j + 3), N, pb=pb)
        if p < 11:
            _chunk_pass(buf, N, [128, 64, 32, 16, 8], scalar_bit=pb)
        else:
            _chunk_pass(buf, N, [128, 64, 32, 16, 8], scalar_bit=pb,
                        post=[0])

    for j in range(11, 4, -1):
        _big_pass(buf, 1 << (j + 3), N)
    _chunk_pass(buf, N, [128, 64, 32, 16, 8], post=[0, 1])

    _chunk_pass(buf, N, [1])
    for j in range(11, 4, -1):
        _big_pass(buf, 1 << (j + 3), N)
    _chunk_pass(buf, N, [128, 64, 32, 16, 8], post=[1, 2])

    _chunk_pass(buf, N, [2, 1])
    for j in range(11, 4, -1):
        _big_pass(buf, 1 << (j + 3), N)
    _chunk_pass(buf, N, [128, 64, 32, 16, 8], post=[2])

    _compact(buf)
    _chunk_pass(buf, TOPK, [2, 1])
    for j in range(11, 5, -1):
        _big_pass(buf, 1 << (j + 2), TOPK)
    _chunk_pass(buf, TOPK, [128, 64, 32, 16, 8, 4])
    _unpermute(buf)

    cout = pltpu.make_async_copy(
        buf.at[pl.ds(TOPK, TOPK), :], o_hbm.at[:, pl.ds(i * CB, CB)],
        sem_out)
    cout.start()
    cout.wait()


def _topk_columns(xt):
    n, r = xt.shape
    return pl.pallas_call(
        _sort_body,
        grid=(r // CB,),
        in_specs=[pl.BlockSpec(memory_space=pl.ANY)],
        out_specs=pl.BlockSpec(memory_space=pl.ANY),
        out_shape=jax.ShapeDtypeStruct((n // 2, r), jnp.float32),
        scratch_shapes=[
            pltpu.VMEM((N, CB), jnp.float32),
            pltpu.SemaphoreType.DMA,
            pltpu.SemaphoreType.DMA,
        ],
        compiler_params=pltpu.CompilerParams(
            dimension_semantics=("arbitrary",),
        ),
    )(xt)


@jax.jit
def kernel(x, layer_idx):
    b, ch, n = x.shape
    x2d = x.reshape(b * ch, n)
    xb = lax.bitcast_convert_type(x2d[:SC_ROWS], jnp.int32)
    out_sc = lax.bitcast_convert_type(_sc_topk(xb), jnp.float32)
    xt = x2d[SC_ROWS:].T
    out_tc = _topk_columns(xt).T
    out = jnp.concatenate([out_sc, out_tc], axis=0).reshape(b, ch, n // 2)
    return out + jnp.zeros((), dtype=out.dtype) * layer_idx


# speedup vs baseline: 5.4483x; 1.2349x over previous
import functools
import jax
import jax.numpy as jnp
from jax import lax
from jax.experimental import pallas as pl
from jax.experimental.pallas import tpu as pltpu
from jax.experimental.pallas import tpu_sc as plsc

SC_N = 32768
SC_K = 16384
SC_ROWS = 128
SC_NV = SC_N // 16
SC_NW = 32
SC_RPW = SC_ROWS // SC_NW


def _key(bits):
    m = lax.shift_right_arithmetic(bits, 31)
    return bits ^ ((m ^ -1) & 0x7FFFFFFF)


def _sc_body(x_hbm, out_hbm, bufa, bufb, bufc, hist):
    lane = lax.iota(jnp.int32, 16)
    ones = jnp.ones((16,), jnp.int32)
    wid = lax.axis_index("s") * 2 + lax.axis_index("c")

    def radix_pass(src, dst, shift, transform):
        def cl(i, carry):
            hist[pl.ds(i * 16, 16)] = jnp.zeros((16,), jnp.int32)
            return carry
        lax.fori_loop(0, 256, cl, 0, unroll=8)

        def ha(v, carry):
            k = src[pl.ds(v * 16, 16)]
            if transform:
                k = _key(k)
            dig = lax.shift_right_logical(k, shift) & 0xFF
            idx = dig * 16 + lane
            cnt = plsc.load_gather(hist, [idx])
            plsc.store_scatter(hist, [idx], cnt + ones)
            return carry
        lax.fori_loop(0, SC_NV, ha, 0, unroll=8)

        def pf(b, s_carry):
            row = hist[pl.ds(b * 16, 16)]
            csum = plsc.cumsum(row)
            tot = jnp.max(csum)
            hist[pl.ds(b * 16, 16)] = (csum - row) + s_carry
            return s_carry + tot
        lax.fori_loop(0, 256, pf, jnp.int32(0), unroll=4)

        def pm(v, carry):
            k = src[pl.ds(v * 16, 16)]
            if transform:
                k = _key(k)
            dig = lax.shift_right_logical(k, shift) & 0xFF
            idx = dig * 16 + lane
            pos = plsc.load_gather(hist, [idx])
            plsc.store_scatter(hist, [idx], pos + 1)
            didx = ((pos & (SC_NV - 1)) << 4) | lax.shift_right_logical(pos, 11)
            plsc.store_scatter(dst, [didx], k)
            return carry
        lax.fori_loop(0, SC_NV, pm, 0, unroll=8)

    def row_loop(ri, carry):
        r = wid * SC_RPW + ri
        pltpu.sync_copy(x_hbm.at[r], bufa)
        radix_pass(bufa, bufb, 0, True)
        radix_pass(bufb, bufc, 8, False)
        radix_pass(bufc, bufb, 16, False)
        radix_pass(bufb, bufc, 24, False)

        def go(j, c2):
            jj = j * 16 + lane
            src_idx = ((jj & (SC_NV - 1)) << 4) | lax.shift_right_logical(jj, 11)
            k = plsc.load_gather(bufc, [src_idx])
            bufa[pl.ds(j * 16, 16)] = _key(k)
            return c2
        lax.fori_loop(0, SC_K // 16, go, 0, unroll=8)
        pltpu.sync_copy(bufa.at[pl.ds(0, SC_K)], out_hbm.at[r])
        return carry

    lax.fori_loop(0, SC_RPW, row_loop, 0)


@functools.partial(
    pl.kernel,
    out_type=jax.ShapeDtypeStruct((SC_ROWS, SC_K), jnp.int32),
    mesh=plsc.VectorSubcoreMesh(core_axis_name="c", subcore_axis_name="s"),
    scratch_types=[
        pltpu.VMEM((SC_N,), jnp.int32),
        pltpu.VMEM((SC_N,), jnp.int32),
        pltpu.VMEM((SC_N,), jnp.int32),
        pltpu.VMEM((4096,), jnp.int32),
    ],
    compiler_params=pltpu.CompilerParams(needs_layout_passes=False),
)
def _sc_topk(x_hbm, out_hbm, bufa, bufb, bufc, hist):
    _sc_body(x_hbm, out_hbm, bufa, bufb, bufc, hist)


N = 32768
TOPK = 16384
CB = 128
CH = 256


def _xchg(c, steps):
    for s in steps:
        cb = c.reshape(CH // (2 * s), 2, s, CB)
        hi = jnp.maximum(cb[:, 0], cb[:, 1])
        lo = jnp.minimum(cb[:, 0], cb[:, 1])
        c = jnp.concatenate(
            [hi[:, None], lo[:, None]], axis=1).reshape(CH, CB)
    return c


def _sgn_static(bits):
    rows = jax.lax.broadcasted_iota(jnp.int32, (CH, 1), 0)
    acc = rows >> bits[0]
    for b in bits[1:]:
        acc = acc ^ (rows >> b)
    return jnp.where((acc & 1) == 0, 1.0, -1.0).astype(jnp.float32)


def _chunk_pass(buf, nr, steps, pre=None, post=None,
                scalar_bit=None, base=0):
    def body(t, carry):
        r0 = base + t * CH
        c = buf[pl.ds(r0, CH), :]
        if scalar_bit is not None:
            sc = (1 - 2 * ((r0 >> scalar_bit) & 1)).astype(jnp.float32)
            c = c * sc
        if pre is not None:
            c = c * _sgn_static(pre)
        c = _xchg(c, steps)
        if post is not None:
            c = c * _sgn_static(post)
        if scalar_bit is not None:
            c = c * sc
        buf[pl.ds(r0, CH), :] = c
        return carry

    jax.lax.fori_loop(0, nr // CH, body, 0)


def _big_pass(buf, s, nr, pb=None, base=0):
    j = s.bit_length() - 1
    n_iter = (nr // 2) // CH

    def body(t, carry):
        tb = t * CH
        off = tb & (s - 1)
        rowa = base + (((tb >> j) << (j + 1)) | off)
        a = buf[pl.ds(rowa, CH), :]
        b = buf[pl.ds(rowa + s, CH), :]
        hi = jnp.maximum(a, b)
        lo = jnp.minimum(a, b)
        if pb is None:
            rhi = rowa
            rlo = rowa + s
        else:
            bs = ((rowa >> pb) & 1) << j
            rhi = rowa + bs
            rlo = rowa + s - bs
        buf[pl.ds(rhi, CH), :] = hi
        buf[pl.ds(rlo, CH), :] = lo
        return carry

    jax.lax.fori_loop(0, n_iter, body, 0)


def _big_pass2(buf, s, nr, pb=None, base=0):
    j = s.bit_length() - 1
    h = s >> 1
    n_iter = (nr // 4) // CH

    def body(t, carry):
        tb = t * CH
        off = tb & (h - 1)
        rowa = base + (((tb >> (j - 1)) << (j + 1)) | off)
        c0 = buf[pl.ds(rowa, CH), :]
        c1 = buf[pl.ds(rowa + h, CH), :]
        c2 = buf[pl.ds(rowa + s, CH), :]
        c3 = buf[pl.ds(rowa + s + h, CH), :]
        a0 = jnp.maximum(c0, c2)
        a2 = jnp.minimum(c0, c2)
        a1 = jnp.maximum(c1, c3)
        a3 = jnp.minimum(c1, c3)
        b0 = jnp.maximum(a0, a1)
        b1 = jnp.minimum(a0, a1)
        b2 = jnp.maximum(a2, a3)
        b3 = jnp.minimum(a2, a3)
        if pb is None:
            r0, r1, r2, r3 = rowa, rowa + h, rowa + s, rowa + s + h
        else:
            bit = (rowa >> pb) & 1
            r0 = rowa + bit * (s + h)
            r1 = rowa + h + bit * (s - h)
            r2 = rowa + s + bit * (h - s)
            r3 = rowa + s + h - bit * (s + h)
        buf[pl.ds(r0, CH), :] = b0
        buf[pl.ds(r1, CH), :] = b1
        buf[pl.ds(r2, CH), :] = b2
        buf[pl.ds(r3, CH), :] = b3
        return carry

    jax.lax.fori_loop(0, n_iter, body, 0)


def _big_run(buf, j_list, nr, pb=None, base=0, shift=3):
    idx = 0
    while idx < len(j_list):
        jv = j_list[idx]
        if (idx + 1 < len(j_list) and j_list[idx + 1] == jv - 1
                and (1 << (jv - 1 + shift)) >= CH):
            _big_pass2(buf, 1 << (jv + shift), nr, pb=pb, base=base)
            idx += 2
        else:
            _big_pass(buf, 1 << (jv + shift), nr, pb=pb, base=base)
            idx += 1


def _compact(buf):
    def body(t, carry):
        ci = buf[pl.ds(2 * t * CH, 2 * CH), :]
        cb = ci.reshape(2 * CH // 8, 2, 4, CB)
        m = jnp.maximum(cb[:, 0], cb[:, 1])
        buf[pl.ds(t * CH, CH), :] = m.reshape(CH, CB)
        return carry

    jax.lax.fori_loop(0, TOPK // CH, body, 0)


def _unpermute(buf):
    def body(t, carry):
        g = t >> 4
        w0 = (t << 10) - (g << 14) + g
        win = buf[pl.ds(w0, 4 * CH), :]
        c = win.reshape(CH, 4, CB)[:, 0, :]
        buf[pl.ds(TOPK + t * CH, CH), :] = c
        return carry

    jax.lax.fori_loop(0, TOPK // CH, body, 0)


def _sort_body(x_hbm, o_hbm, buf, sem_in, sem_out):
    i = pl.program_id(0)
    cin = pltpu.make_async_copy(
        x_hbm.at[:, pl.ds(i * CB, CB)], buf, sem_in)
    cin.start()
    cin.wait()

    def s14(t, carry):
        r0 = t * CH
        c = buf[pl.ds(r0, CH), :]
        c = c * _sgn_static([4])
        c = _xchg(c, [8])
        c = c * _sgn_static([4, 5])
        c = _xchg(c, [16, 8])
        c = c * _sgn_static([5, 6])
        c = _xchg(c, [32, 16, 8])
        c = c * _sgn_static([6, 7])
        c = _xchg(c, [64, 32, 16, 8])
        c = c * _sgn_static([7])
        buf[pl.ds(r0, CH), :] = c
        return carry
    jax.lax.fori_loop(0, N // CH, s14, 0)

    for p in range(5, 12):
        pb = p + 3
        _big_run(buf, list(range(p - 1, 4, -1)), N, pb=pb)
        if p < 11:
            _chunk_pass(buf, N, [128, 64, 32, 16, 8], scalar_bit=pb)
        else:
            _chunk_pass(buf, N, [128, 64, 32, 16, 8], scalar_bit=pb,
                        post=[0])

    _big_run(buf, list(range(11, 4, -1)), N)
    _chunk_pass(buf, N, [128, 64, 32, 16, 8], post=[0, 1])

    _chunk_pass(buf, N, [1])
    _big_run(buf, list(range(11, 4, -1)), N)
    _chunk_pass(buf, N, [128, 64, 32, 16, 8], post=[1, 2])

    _chunk_pass(buf, N, [2, 1])
    _big_run(buf, list(range(11, 4, -1)), N)
    _chunk_pass(buf, N, [128, 64, 32, 16, 8], post=[2])

    _compact(buf)
    _chunk_pass(buf, TOPK, [2, 1])
    _big_run(buf, list(range(11, 5, -1)), TOPK, shift=2)
    _chunk_pass(buf, TOPK, [128, 64, 32, 16, 8, 4])
    _unpermute(buf)

    cout = pltpu.make_async_copy(
        buf.at[pl.ds(TOPK, TOPK), :], o_hbm.at[:, pl.ds(i * CB, CB)],
        sem_out)
    cout.start()
    cout.wait()


def _topk_columns(xt):
    n, r = xt.shape
    return pl.pallas_call(
        _sort_body,
        grid=(r // CB,),
        in_specs=[pl.BlockSpec(memory_space=pl.ANY)],
        out_specs=pl.BlockSpec(memory_space=pl.ANY),
        out_shape=jax.ShapeDtypeStruct((n // 2, r), jnp.float32),
        scratch_shapes=[
            pltpu.VMEM((N, CB), jnp.float32),
            pltpu.SemaphoreType.DMA,
            pltpu.SemaphoreType.DMA,
        ],
        compiler_params=pltpu.CompilerParams(
            dimension_semantics=("arbitrary",),
        ),
    )(xt)


@jax.jit
def kernel(x, layer_idx):
    b, ch, n = x.shape
    x2d = x.reshape(b * ch, n)
    xb = lax.bitcast_convert_type(x2d[:SC_ROWS], jnp.int32)
    out_sc = lax.bitcast_convert_type(_sc_topk(xb), jnp.float32)
    xt = x2d[SC_ROWS:].T
    out_tc = _topk_columns(xt).T
    out = jnp.concatenate([out_sc, out_tc], axis=0).reshape(b, ch, n // 2)
    return out + jnp.zeros((), dtype=out.dtype) * layer_idx


# speedup vs baseline: 5.9664x; 1.0951x over previous
import functools
import jax
import jax.numpy as jnp
from jax import lax
from jax.experimental import pallas as pl
from jax.experimental.pallas import tpu as pltpu
from jax.experimental.pallas import tpu_sc as plsc

SC_N = 32768
SC_K = 16384
SC_ROWS = 128
SC_NV = SC_N // 16
SC_NW = 32
SC_RPW = SC_ROWS // SC_NW


def _key(bits):
    m = lax.shift_right_arithmetic(bits, 31)
    return bits ^ ((m ^ -1) & 0x7FFFFFFF)


def _sc_body(x_hbm, out_hbm, bufa, bufb, bufc, hist):
    lane = lax.iota(jnp.int32, 16)
    ones = jnp.ones((16,), jnp.int32)
    wid = lax.axis_index("s") * 2 + lax.axis_index("c")

    def radix_pass(src, dst, shift, transform):
        def cl(i, carry):
            hist[pl.ds(i * 16, 16)] = jnp.zeros((16,), jnp.int32)
            return carry
        lax.fori_loop(0, 256, cl, 0, unroll=8)

        def ha(v, carry):
            k = src[pl.ds(v * 16, 16)]
            if transform:
                k = _key(k)
            dig = lax.shift_right_logical(k, shift) & 0xFF
            idx = dig * 16 + lane
            cnt = plsc.load_gather(hist, [idx])
            plsc.store_scatter(hist, [idx], cnt + ones)
            return carry
        lax.fori_loop(0, SC_NV, ha, 0, unroll=8)

        def pf(b, s_carry):
            row = hist[pl.ds(b * 16, 16)]
            csum = plsc.cumsum(row)
            tot = jnp.max(csum)
            hist[pl.ds(b * 16, 16)] = (csum - row) + s_carry
            return s_carry + tot
        lax.fori_loop(0, 256, pf, jnp.int32(0), unroll=4)

        def pm(v, carry):
            k = src[pl.ds(v * 16, 16)]
            if transform:
                k = _key(k)
            dig = lax.shift_right_logical(k, shift) & 0xFF
            idx = dig * 16 + lane
            pos = plsc.load_gather(hist, [idx])
            plsc.store_scatter(hist, [idx], pos + 1)
            didx = ((pos & (SC_NV - 1)) << 4) | lax.shift_right_logical(pos, 11)
            plsc.store_scatter(dst, [didx], k)
            return carry
        lax.fori_loop(0, SC_NV, pm, 0, unroll=8)

    def row_loop(ri, carry):
        r = wid * SC_RPW + ri
        pltpu.sync_copy(x_hbm.at[r], bufa)
        radix_pass(bufa, bufb, 0, True)
        radix_pass(bufb, bufc, 8, False)
        radix_pass(bufc, bufb, 16, False)
        radix_pass(bufb, bufc, 24, False)

        def go(j, c2):
            jj = j * 16 + lane
            src_idx = ((jj & (SC_NV - 1)) << 4) | lax.shift_right_logical(jj, 11)
            k = plsc.load_gather(bufc, [src_idx])
            bufa[pl.ds(j * 16, 16)] = _key(k)
            return c2
        lax.fori_loop(0, SC_K // 16, go, 0, unroll=8)
        pltpu.sync_copy(bufa.at[pl.ds(0, SC_K)], out_hbm.at[r])
        return carry

    lax.fori_loop(0, SC_RPW, row_loop, 0)


@functools.partial(
    pl.kernel,
    out_type=jax.ShapeDtypeStruct((SC_ROWS, SC_K), jnp.int32),
    mesh=plsc.VectorSubcoreMesh(core_axis_name="c", subcore_axis_name="s"),
    scratch_types=[
        pltpu.VMEM((SC_N,), jnp.int32),
        pltpu.VMEM((SC_N,), jnp.int32),
        pltpu.VMEM((SC_N,), jnp.int32),
        pltpu.VMEM((4096,), jnp.int32),
    ],
    compiler_params=pltpu.CompilerParams(needs_layout_passes=False),
)
def _sc_topk(x_hbm, out_hbm, bufa, bufb, bufc, hist):
    _sc_body(x_hbm, out_hbm, bufa, bufb, bufc, hist)


N = 32768
TOPK = 16384
CB = 128
CH = 256


def _xchg(c, steps):
    for s in steps:
        cb = c.reshape(CH // (2 * s), 2, s, CB)
        hi = jnp.maximum(cb[:, 0], cb[:, 1])
        lo = jnp.minimum(cb[:, 0], cb[:, 1])
        c = jnp.concatenate(
            [hi[:, None], lo[:, None]], axis=1).reshape(CH, CB)
    return c


def _sgn_static(bits):
    rows = jax.lax.broadcasted_iota(jnp.int32, (CH, 1), 0)
    acc = rows >> bits[0]
    for b in bits[1:]:
        acc = acc ^ (rows >> b)
    return jnp.where((acc & 1) == 0, 1.0, -1.0).astype(jnp.float32)


def _chunk_pass(buf, nr, steps, pre=None, post=None,
                scalar_bit=None, base=0):
    pre_m = _sgn_static(pre) if pre is not None else None
    post_m = _sgn_static(post) if post is not None else None

    def body(t, carry):
        r0 = base + t * CH
        c = buf[pl.ds(r0, CH), :]
        if scalar_bit is not None:
            sc = (1 - 2 * ((r0 >> scalar_bit) & 1)).astype(jnp.float32)
            c = c * sc
        if pre_m is not None:
            c = c * pre_m
        c = _xchg(c, steps)
        if post_m is not None:
            c = c * post_m
        if scalar_bit is not None:
            c = c * sc
        buf[pl.ds(r0, CH), :] = c
        return carry

    jax.lax.fori_loop(0, nr // CH, body, 0)


def _big_pass(buf, s, nr, pb=None, base=0):
    j = s.bit_length() - 1
    n_iter = (nr // 2) // CH

    def body(t, carry):
        tb = t * CH
        off = tb & (s - 1)
        rowa = base + (((tb >> j) << (j + 1)) | off)
        a = buf[pl.ds(rowa, CH), :]
        b = buf[pl.ds(rowa + s, CH), :]
        hi = jnp.maximum(a, b)
        lo = jnp.minimum(a, b)
        if pb is None:
            rhi = rowa
            rlo = rowa + s
        else:
            bs = ((rowa >> pb) & 1) << j
            rhi = rowa + bs
            rlo = rowa + s - bs
        buf[pl.ds(rhi, CH), :] = hi
        buf[pl.ds(rlo, CH), :] = lo
        return carry

    jax.lax.fori_loop(0, n_iter, body, 0)


def _big_pass2(buf, s, nr, pb=None, base=0):
    j = s.bit_length() - 1
    h = s >> 1
    n_iter = (nr // 4) // CH

    def body(t, carry):
        tb = t * CH
        off = tb & (h - 1)
        rowa = base + (((tb >> (j - 1)) << (j + 1)) | off)
        c0 = buf[pl.ds(rowa, CH), :]
        c1 = buf[pl.ds(rowa + h, CH), :]
        c2 = buf[pl.ds(rowa + s, CH), :]
        c3 = buf[pl.ds(rowa + s + h, CH), :]
        a0 = jnp.maximum(c0, c2)
        a2 = jnp.minimum(c0, c2)
        a1 = jnp.maximum(c1, c3)
        a3 = jnp.minimum(c1, c3)
        b0 = jnp.maximum(a0, a1)
        b1 = jnp.minimum(a0, a1)
        b2 = jnp.maximum(a2, a3)
        b3 = jnp.minimum(a2, a3)
        if pb is None:
            r0, r1, r2, r3 = rowa, rowa + h, rowa + s, rowa + s + h
        else:
            bit = (rowa >> pb) & 1
            r0 = rowa + bit * (s + h)
            r1 = rowa + h + bit * (s - h)
            r2 = rowa + s + bit * (h - s)
            r3 = rowa + s + h - bit * (s + h)
        buf[pl.ds(r0, CH), :] = b0
        buf[pl.ds(r1, CH), :] = b1
        buf[pl.ds(r2, CH), :] = b2
        buf[pl.ds(r3, CH), :] = b3
        return carry

    jax.lax.fori_loop(0, n_iter, body, 0)


def _big_run(buf, j_list, nr, pb=None, base=0, shift=3):
    idx = 0
    while idx < len(j_list):
        jv = j_list[idx]
        if (idx + 1 < len(j_list) and j_list[idx + 1] == jv - 1
                and (1 << (jv - 1 + shift)) >= CH):
            _big_pass2(buf, 1 << (jv + shift), nr, pb=pb, base=base)
            idx += 2
        else:
            _big_pass(buf, 1 << (jv + shift), nr, pb=pb, base=base)
            idx += 1


def _compact(buf):
    def body(t, carry):
        ci = buf[pl.ds(2 * t * CH, 2 * CH), :]
        cb = ci.reshape(2 * CH // 8, 2, 4, CB)
        m = jnp.maximum(cb[:, 0], cb[:, 1])
        buf[pl.ds(t * CH, CH), :] = m.reshape(CH, CB)
        return carry

    jax.lax.fori_loop(0, TOPK // CH, body, 0)


def _unpermute(buf):
    def body(t, carry):
        g = t >> 4
        w0 = (t << 10) - (g << 14) + g
        win = buf[pl.ds(w0, 4 * CH), :]
        c = win.reshape(CH, 4, CB)[:, 0, :]
        buf[pl.ds(TOPK + t * CH, CH), :] = c
        return carry

    jax.lax.fori_loop(0, TOPK // CH, body, 0)


def _sort_body(x_hbm, o_hbm, buf, sem_in, sem_out):
    i = pl.program_id(0)
    cin = pltpu.make_async_copy(
        x_hbm.at[:, pl.ds(i * CB, CB)], buf, sem_in)
    cin.start()
    cin.wait()

    m1 = _sgn_static([4])
    m12 = _sgn_static([4, 5])
    m23 = _sgn_static([5, 6])
    m34 = _sgn_static([6, 7])
    m4 = _sgn_static([7])

    def s14(t, carry):
        r0 = t * CH
        c = buf[pl.ds(r0, CH), :]
        c = c * m1
        c = _xchg(c, [8])
        c = c * m12
        c = _xchg(c, [16, 8])
        c = c * m23
        c = _xchg(c, [32, 16, 8])
        c = c * m34
        c = _xchg(c, [64, 32, 16, 8])
        c = c * m4
        buf[pl.ds(r0, CH), :] = c
        return carry
    jax.lax.fori_loop(0, N // CH, s14, 0)

    for p in range(5, 12):
        pb = p + 3
        _big_run(buf, list(range(p - 1, 4, -1)), N, pb=pb)
        if p < 11:
            _chunk_pass(buf, N, [128, 64, 32, 16, 8], scalar_bit=pb)
        else:
            _chunk_pass(buf, N, [128, 64, 32, 16, 8], scalar_bit=pb,
                        post=[0])

    _big_run(buf, list(range(11, 4, -1)), N)
    _chunk_pass(buf, N, [128, 64, 32, 16, 8], post=[0, 1])

    _chunk_pass(buf, N, [1])
    _big_run(buf, list(range(11, 4, -1)), N)
    _chunk_pass(buf, N, [128, 64, 32, 16, 8], post=[1, 2])

    _chunk_pass(buf, N, [2, 1])
    _big_run(buf, list(range(11, 4, -1)), N)
    _chunk_pass(buf, N, [128, 64, 32, 16, 8], post=[2])

    _compact(buf)
    _chunk_pass(buf, TOPK, [2, 1])
    _big_run(buf, list(range(11, 5, -1)), TOPK, shift=2)
    _chunk_pass(buf, TOPK, [128, 64, 32, 16, 8, 4])
    _unpermute(buf)

    cout = pltpu.make_async_copy(
        buf.at[pl.ds(TOPK, TOPK), :], o_hbm.at[:, pl.ds(i * CB, CB)],
        sem_out)
    cout.start()
    cout.wait()


def _topk_columns(xt):
    n, r = xt.shape
    return pl.pallas_call(
        _sort_body,
        grid=(r // CB,),
        in_specs=[pl.BlockSpec(memory_space=pl.ANY)],
        out_specs=pl.BlockSpec(memory_space=pl.ANY),
        out_shape=jax.ShapeDtypeStruct((n // 2, r), jnp.float32),
        scratch_shapes=[
            pltpu.VMEM((N, CB), jnp.float32),
            pltpu.SemaphoreType.DMA,
            pltpu.SemaphoreType.DMA,
        ],
        compiler_params=pltpu.CompilerParams(
            dimension_semantics=("arbitrary",),
        ),
    )(xt)


@jax.jit
def kernel(x, layer_idx):
    b, ch, n = x.shape
    x2d = x.reshape(b * ch, n)
    xb = lax.bitcast_convert_type(x2d[:SC_ROWS], jnp.int32)
    out_sc = lax.bitcast_convert_type(_sc_topk(xb), jnp.float32)
    xt = x2d[SC_ROWS:].T
    out_tc = _topk_columns(xt).T
    out = jnp.concatenate([out_sc, out_tc], axis=0).reshape(b, ch, n // 2)
    return out + jnp.zeros((), dtype=out.dtype) * layer_idx
